# baseline (speedup 1.0000x reference)
import functools

import jax
import jax.numpy as jnp
from jax import lax
from jax.experimental import pallas as pl
from jax.experimental.pallas import tpu as pltpu
from jax.experimental.pallas import tpu_sc as plsc

C = 64
K = 16
L = 16
NC = 2
NS = 16
NW = NC * NS


def _edac_body(rows_w, rows_c, nchunks,
               main_hbm, dup_hbm, minv_hbm, maxv_hbm, out_hbm,
               m0, m1, d0, d1, mn_ref, mx_ref,
               in_s0, in_s1, out_s0, out_s1):
    w = lax.axis_index("s") * NC + lax.axis_index("c")
    row0 = w * rows_w

    pltpu.sync_copy(minv_hbm, mn_ref)
    pltpu.sync_copy(maxv_hbm, mx_ref)

    idx4 = lax.iota(jnp.int32, L) * 4
    minv4 = plsc.load_gather(mn_ref, [idx4])
    maxv4 = plsc.load_gather(mx_ref, [idx4])
    mn_p = [mn_ref[pl.ds(p * L, L)] for p in range(4)]
    mx_p = [mx_ref[pl.ds(p * L, L)] for p in range(4)]

    mbufs = [m0, m1]
    dbufs = [d0, d1]
    in_sems = [in_s0, in_s1]
    out_sems = [out_s0, out_s1]
    in_cps = [None, None]
    out_cps = [None, None]

    def start_load(i):
        b = i % 2
        off = row0 + i * rows_c
        cpm = pltpu.async_copy(
            main_hbm.at[pl.ds(off * C, rows_c * C)], mbufs[b], in_sems[b])
        cpd = pltpu.async_copy(
            dup_hbm.at[pl.ds(off * K, rows_c * K)], dbufs[b], in_sems[b])
        in_cps[b] = (cpm, cpd)

    def compute(mb, db):
        def row_body(r, carry):
            base = r * C
            gidx = base + idx4
            d = db[pl.ds(r * K, K)]
            m = plsc.load_gather(mb, [gidx])
            mval = (m >= minv4) & (m <= maxv4)
            dval = (d >= minv4) & (d <= maxv4)
            mbig = jnp.where(mval, m, jnp.float32(jnp.inf))
            dbig = jnp.where(dval, d, jnp.float32(jnp.inf))
            v = jnp.minimum(mbig, dbig)
            v = jnp.where(mval | dval, v, jnp.float32(0.0))
            plsc.store_scatter(mb, [gidx], v)
            for p in range(4):
                sl = pl.ds(base + p * L, L)
                x = mb[sl]
                keep = (x >= mn_p[p]) & (x <= mx_p[p])
                mb[sl] = jnp.where(keep, x, jnp.float32(0.0))
            return carry

        lax.fori_loop(0, rows_c, row_body, 0, unroll=2)

    start_load(0)
    for i in range(nchunks):
        b = i % 2
        if i + 1 < nchunks:
            nb = (i + 1) % 2
            if i >= 1:
                out_cps[nb].wait()
            start_load(i + 1)
        in_cps[b][0].wait()
        in_cps[b][1].wait()
        compute(mbufs[b], dbufs[b])
        out_cps[b] = pltpu.async_copy(
            mbufs[b], out_hbm.at[pl.ds((row0 + i * rows_c) * C, rows_c * C)],
            out_sems[b])
    out_cps[(nchunks - 1) % 2].wait()
    if nchunks >= 2:
        out_cps[nchunks % 2].wait()


def kernel(main_out, dup_out, min_vals, max_vals):
    B = main_out.shape[0]
    rows_w = B // NW
    rows_c = min(rows_w, 256)
    nchunks = rows_w // rows_c

    mesh = plsc.VectorSubcoreMesh(core_axis_name="c", subcore_axis_name="s")
    body = functools.partial(_edac_body, rows_w, rows_c, nchunks)
    f = pl.kernel(
        body,
        out_type=jax.ShapeDtypeStruct((B * C,), jnp.float32),
        mesh=mesh,
        compiler_params=pltpu.CompilerParams(needs_layout_passes=False),
        scratch_types=[
            pltpu.VMEM((rows_c * C,), jnp.float32),
            pltpu.VMEM((rows_c * C,), jnp.float32),
            pltpu.VMEM((rows_c * K,), jnp.float32),
            pltpu.VMEM((rows_c * K,), jnp.float32),
            pltpu.VMEM((C,), jnp.float32),
            pltpu.VMEM((C,), jnp.float32),
            pltpu.SemaphoreType.DMA,
            pltpu.SemaphoreType.DMA,
            pltpu.SemaphoreType.DMA,
            pltpu.SemaphoreType.DMA,
        ],
    )
    out_flat = f(main_out.reshape(-1), dup_out.reshape(-1),
                 min_vals, max_vals)
    return out_flat.reshape(B, C)


# speedup vs baseline: 1.0905x; 1.0905x over previous
import functools

import jax
import jax.numpy as jnp
from jax import lax
from jax.experimental import pallas as pl
from jax.experimental.pallas import tpu as pltpu
from jax.experimental.pallas import tpu_sc as plsc

C = 64
K = 16
L = 16
NC = 2
NS = 16
NW = NC * NS


def _edac_body(rows_w, rows_c, nchunks,
               main_hbm, dup_hbm, minv_hbm, maxv_hbm, out_hbm,
               m0, m1, d0, d1, mn_ref, mx_ref,
               in_s0, in_s1, out_s0, out_s1):
    w = lax.axis_index("s") * NC + lax.axis_index("c")
    row0 = w * rows_w

    pltpu.sync_copy(minv_hbm, mn_ref)
    pltpu.sync_copy(maxv_hbm, mx_ref)

    idx4 = lax.iota(jnp.int32, L) * 4
    minv4 = plsc.load_gather(mn_ref, [idx4])
    maxv4 = plsc.load_gather(mx_ref, [idx4])
    mn_p = [mn_ref[pl.ds(p * L, L)] for p in range(4)]
    mx_p = [mx_ref[pl.ds(p * L, L)] for p in range(4)]

    mbufs = [m0, m1]
    dbufs = [d0, d1]
    in_sems = [in_s0, in_s1]
    out_sems = [out_s0, out_s1]
    in_cps = [None, None]
    out_cps = [None, None]

    def start_load(i):
        b = i % 2
        off = row0 + i * rows_c
        cpm = pltpu.async_copy(
            main_hbm.at[pl.ds(off * C, rows_c * C)], mbufs[b], in_sems[b])
        cpd = pltpu.async_copy(
            dup_hbm.at[pl.ds(off * K, rows_c * K)], dbufs[b], in_sems[b])
        in_cps[b] = (cpm, cpd)

    def compute(mb, db):
        @plsc.parallel_loop(0, rows_c, unroll=4)
        def row_body(r):
            base = r * C
            gidx = base + idx4
            d = db[pl.ds(r * K, K)]
            m = plsc.load_gather(mb, [gidx])
            mval = (m >= minv4) & (m <= maxv4)
            dval = (d >= minv4) & (d <= maxv4)
            mbig = jnp.where(mval, m, jnp.float32(jnp.inf))
            dbig = jnp.where(dval, d, jnp.float32(jnp.inf))
            v = jnp.minimum(mbig, dbig)
            v = jnp.where(mval | dval, v, jnp.float32(0.0))
            plsc.store_scatter(mb, [gidx], v)
            for p in range(4):
                sl = pl.ds(base + p * L, L)
                x = mb[sl]
                keep = (x >= mn_p[p]) & (x <= mx_p[p])
                mb[sl] = jnp.where(keep, x, jnp.float32(0.0))

    start_load(0)
    for i in range(nchunks):
        b = i % 2
        if i + 1 < nchunks:
            nb = (i + 1) % 2
            if i >= 1:
                out_cps[nb].wait()
            start_load(i + 1)
        in_cps[b][0].wait()
        in_cps[b][1].wait()
        compute(mbufs[b], dbufs[b])
        out_cps[b] = pltpu.async_copy(
            mbufs[b], out_hbm.at[pl.ds((row0 + i * rows_c) * C, rows_c * C)],
            out_sems[b])
    out_cps[(nchunks - 1) % 2].wait()
    if nchunks >= 2:
        out_cps[nchunks % 2].wait()


def kernel(main_out, dup_out, min_vals, max_vals):
    B = main_out.shape[0]
    rows_w = B // NW
    rows_c = min(rows_w, 256)
    nchunks = rows_w // rows_c

    mesh = plsc.VectorSubcoreMesh(core_axis_name="c", subcore_axis_name="s")
    body = functools.partial(_edac_body, rows_w, rows_c, nchunks)
    f = pl.kernel(
        body,
        out_type=jax.ShapeDtypeStruct((B * C,), jnp.float32),
        mesh=mesh,
        compiler_params=pltpu.CompilerParams(needs_layout_passes=False),
        scratch_types=[
            pltpu.VMEM((rows_c * C,), jnp.float32),
            pltpu.VMEM((rows_c * C,), jnp.float32),
            pltpu.VMEM((rows_c * K,), jnp.float32),
            pltpu.VMEM((rows_c * K,), jnp.float32),
            pltpu.VMEM((C,), jnp.float32),
            pltpu.VMEM((C,), jnp.float32),
            pltpu.SemaphoreType.DMA,
            pltpu.SemaphoreType.DMA,
            pltpu.SemaphoreType.DMA,
            pltpu.SemaphoreType.DMA,
        ],
    )
    out_flat = f(main_out.reshape(-1), dup_out.reshape(-1),
                 min_vals, max_vals)
    return out_flat.reshape(B, C)


# speedup vs baseline: 3.8192x; 3.5022x over previous
import functools

import jax
import jax.numpy as jnp
from jax import lax
from jax.experimental import pallas as pl
from jax.experimental.pallas import tpu as pltpu
from jax.experimental.pallas import tpu_sc as plsc

C = 64
K = 16
L = 16
NC = 2
NS = 16
NW = NC * NS
TB = 128
CB = 8
NQ = 4


def _edac_body(tc_w, tc_c, nchunks,
               main_hbm, dup_hbm, minv_hbm, maxv_hbm, out_hbm,
               m0, m1, d0, d1, mn_ref, mx_ref,
               in_s0, in_s1, out_s0, out_s1):
    w = lax.axis_index("s") * NC + lax.axis_index("c")
    tr = w // NQ
    q = w % NQ
    tc0 = q * tc_w

    pltpu.sync_copy(minv_hbm, mn_ref)
    pltpu.sync_copy(maxv_hbm, mx_ref)

    mn_vecs = []
    mx_vecs = []
    for r in range(CB):
        ch = jnp.full((L,), tr * CB + r, dtype=jnp.int32)
        mn_vecs.append(plsc.load_gather(mn_ref, [ch]))
        mx_vecs.append(plsc.load_gather(mx_ref, [ch]))

    trd = tr // 4
    rd = (tr * 2) % CB

    mbufs = [m0, m1]
    dbufs = [d0, d1]
    in_sems = [in_s0, in_s1]
    out_sems = [out_s0, out_s1]
    in_cps = [None, None]
    out_cps = [None, None]

    def start_load(i):
        b = i % 2
        t0 = tc0 + i * tc_c
        cpm = pltpu.async_copy(
            main_hbm.at[tr, pl.ds(t0, tc_c), :, :], mbufs[b], in_sems[b])
        cpd = pltpu.async_copy(
            dup_hbm.at[trd, pl.ds(t0, tc_c), pl.ds(rd, 2), :],
            dbufs[b], in_sems[b])
        in_cps[b] = (cpm, cpd)

    inf = jnp.float32(jnp.inf)
    zero = jnp.float32(0.0)

    def compute(mb, db):
        @plsc.parallel_loop(0, tc_c, unroll=2)
        def tile_body(t):
            for r in range(CB):
                mnv = mn_vecs[r]
                mxv = mx_vecs[r]
                if r % 4 == 0:
                    rr = r // 4
                    for j in range(TB // L):
                        sl = pl.ds(j * L, L)
                        m = mb[t, r, sl]
                        d = db[t, rr, sl]
                        mval = (m >= mnv) & (m <= mxv)
                        dval = (d >= mnv) & (d <= mxv)
                        v = jnp.minimum(jnp.where(mval, m, inf),
                                        jnp.where(dval, d, inf))
                        mb[t, r, sl] = jnp.where(mval | dval, v, zero)
                else:
                    for j in range(TB // L):
                        sl = pl.ds(j * L, L)
                        x = mb[t, r, sl]
                        keep = (x >= mnv) & (x <= mxv)
                        mb[t, r, sl] = jnp.where(keep, x, zero)

    start_load(0)
    for i in range(nchunks):
        b = i % 2
        if i + 1 < nchunks:
            nb = (i + 1) % 2
            if i >= 1:
                out_cps[nb].wait()
            start_load(i + 1)
        in_cps[b][0].wait()
        in_cps[b][1].wait()
        compute(mbufs[b], dbufs[b])
        out_cps[b] = pltpu.async_copy(
            mbufs[b], out_hbm.at[tr, pl.ds(tc0 + i * tc_c, tc_c), :, :],
            out_sems[b])
    out_cps[(nchunks - 1) % 2].wait()
    if nchunks >= 2:
        out_cps[nchunks % 2].wait()


def kernel(main_out, dup_out, min_vals, max_vals):
    B = main_out.shape[0]
    nt = B // TB
    tc_w = nt // NQ
    tc_c = min(tc_w, 16)
    nchunks = tc_w // tc_c

    main4 = main_out.reshape(nt, TB, CB, CB).transpose(2, 0, 3, 1)
    dup4 = dup_out.reshape(nt, TB, K // CB, CB).transpose(2, 0, 3, 1)

    mesh = plsc.VectorSubcoreMesh(core_axis_name="c", subcore_axis_name="s")
    body = functools.partial(_edac_body, tc_w, tc_c, nchunks)
    f = pl.kernel(
        body,
        out_type=jax.ShapeDtypeStruct((CB, nt, CB, TB), jnp.float32),
        mesh=mesh,
        compiler_params=pltpu.CompilerParams(needs_layout_passes=False),
        scratch_types=[
            pltpu.VMEM((tc_c, CB, TB), jnp.float32),
            pltpu.VMEM((tc_c, CB, TB), jnp.float32),
            pltpu.VMEM((tc_c, 2, TB), jnp.float32),
            pltpu.VMEM((tc_c, 2, TB), jnp.float32),
            pltpu.VMEM((C,), jnp.float32),
            pltpu.VMEM((C,), jnp.float32),
            pltpu.SemaphoreType.DMA,
            pltpu.SemaphoreType.DMA,
            pltpu.SemaphoreType.DMA,
            pltpu.SemaphoreType.DMA,
        ],
    )
    out4 = f(main4, dup4, min_vals, max_vals)
    return out4.transpose(1, 3, 0, 2).reshape(B, C)


# speedup vs baseline: 3.9732x; 1.0403x over previous
import functools

import jax
import jax.numpy as jnp
from jax import lax
from jax.experimental import pallas as pl
from jax.experimental.pallas import tpu as pltpu
from jax.experimental.pallas import tpu_sc as plsc

C = 64
K = 16
L = 16
NC = 2
NS = 16
NW = NC * NS
TB = 128
CB = 8
NQ = 4


def _edac_body(tc_w, tc_c, nchunks,
               main_hbm, dup_hbm, minv_hbm, maxv_hbm, out_hbm,
               m0, m1, d0, d1, mn_ref, mx_ref,
               in_s0, in_s1, out_s0, out_s1):
    w = lax.axis_index("s") * NC + lax.axis_index("c")
    tr = w // NQ
    q = w % NQ
    tc0 = q * tc_w

    pltpu.sync_copy(minv_hbm, mn_ref)
    pltpu.sync_copy(maxv_hbm, mx_ref)

    mn_vecs = []
    mx_vecs = []
    for r in range(CB):
        ch = jnp.full((L,), tr * CB + r, dtype=jnp.int32)
        mn_vecs.append(plsc.load_gather(mn_ref, [ch]))
        mx_vecs.append(plsc.load_gather(mx_ref, [ch]))

    trd = tr // 4
    rd = (tr * 2) % CB

    mbufs = [m0, m1]
    dbufs = [d0, d1]
    in_sems = [in_s0, in_s1]
    out_sems = [out_s0, out_s1]
    in_cps = [None, None]
    out_cps = [None, None]

    def start_load(i):
        b = i % 2
        t0 = tc0 + i * tc_c
        cpm = pltpu.async_copy(
            main_hbm.at[tr, pl.ds(t0, tc_c), :, :], mbufs[b], in_sems[b])
        cpd = pltpu.async_copy(
            dup_hbm.at[trd, pl.ds(t0, tc_c), pl.ds(rd, 2), :],
            dbufs[b], in_sems[b])
        in_cps[b] = (cpm, cpd)

    inf = jnp.float32(jnp.inf)
    zero = jnp.float32(0.0)

    def compute(mb, db):
        @plsc.parallel_loop(0, tc_c, unroll=2)
        def tile_body(t):
            for r in range(CB):
                mnv = mn_vecs[r]
                mxv = mx_vecs[r]
                if r % 4 == 0:
                    rr = r // 4
                    for j in range(TB // L):
                        sl = pl.ds(j * L, L)
                        m = mb[t, r, sl]
                        d = db[t, rr, sl]
                        mval = (m >= mnv) & (m <= mxv)
                        dval = (d >= mnv) & (d <= mxv)
                        v = jnp.minimum(jnp.where(mval, m, inf),
                                        jnp.where(dval, d, inf))
                        mb[t, r, sl] = jnp.where(mval | dval, v, zero)
                else:
                    for j in range(TB // L):
                        sl = pl.ds(j * L, L)
                        x = mb[t, r, sl]
                        keep = (x >= mnv) & (x <= mxv)
                        mb[t, r, sl] = jnp.where(keep, x, zero)

    start_load(0)
    for i in range(nchunks):
        b = i % 2
        if i + 1 < nchunks:
            nb = (i + 1) % 2
            if i >= 1:
                out_cps[nb].wait()
            start_load(i + 1)
        in_cps[b][0].wait()
        in_cps[b][1].wait()
        compute(mbufs[b], dbufs[b])
        out_cps[b] = pltpu.async_copy(
            mbufs[b], out_hbm.at[tr, pl.ds(tc0 + i * tc_c, tc_c), :, :],
            out_sems[b])
    out_cps[(nchunks - 1) % 2].wait()
    if nchunks >= 2:
        out_cps[nchunks % 2].wait()


def kernel(main_out, dup_out, min_vals, max_vals):
    B = main_out.shape[0]
    nt = B // TB
    tc_w = nt // NQ
    tc_c = min(tc_w, 32)
    nchunks = tc_w // tc_c

    main4 = main_out.reshape(nt, TB, CB, CB).transpose(2, 0, 3, 1)
    dup4 = dup_out.reshape(nt, TB, K // CB, CB).transpose(2, 0, 3, 1)

    mesh = plsc.VectorSubcoreMesh(core_axis_name="c", subcore_axis_name="s")
    body = functools.partial(_edac_body, tc_w, tc_c, nchunks)
    f = pl.kernel(
        body,
        out_type=jax.ShapeDtypeStruct((CB, nt, CB, TB), jnp.float32),
        mesh=mesh,
        compiler_params=pltpu.CompilerParams(needs_layout_passes=False),
        scratch_types=[
            pltpu.VMEM((tc_c, CB, TB), jnp.float32),
            pltpu.VMEM((tc_c, CB, TB), jnp.float32),
            pltpu.VMEM((tc_c, 2, TB), jnp.float32),
            pltpu.VMEM((tc_c, 2, TB), jnp.float32),
            pltpu.VMEM((C,), jnp.float32),
            pltpu.VMEM((C,), jnp.float32),
            pltpu.SemaphoreType.DMA,
            pltpu.SemaphoreType.DMA,
            pltpu.SemaphoreType.DMA,
            pltpu.SemaphoreType.DMA,
        ],
    )
    out4 = f(main4, dup4, min_vals, max_vals)
    return out4.transpose(1, 3, 0, 2).reshape(B, C)


# speedup vs baseline: 4.4216x; 1.1129x over previous
import functools

import jax
import jax.numpy as jnp
from jax import lax
from jax.experimental import pallas as pl
from jax.experimental.pallas import tpu as pltpu
from jax.experimental.pallas import tpu_sc as plsc

C = 64
K = 16
L = 16
NC = 2
NS = 16
NW = NC * NS
TB = 128
CB = 8
NQ = 4


def _edac_body(tc_w, tc_c, nchunks,
               main_hbm, dup_hbm, minv_hbm, maxv_hbm, out_hbm,
               m0, m1, d0, d1, mn_ref, mx_ref,
               in_s0, in_s1, out_s0, out_s1):
    w = lax.axis_index("s") * NC + lax.axis_index("c")
    tr = w // NQ
    q = w % NQ
    tc0 = q * tc_w

    pltpu.sync_copy(minv_hbm, mn_ref)
    pltpu.sync_copy(maxv_hbm, mx_ref)

    mn_vecs = []
    mx_vecs = []
    for r in range(CB):
        ch = jnp.full((L,), tr * CB + r, dtype=jnp.int32)
        mn_vecs.append(plsc.load_gather(mn_ref, [ch]))
        mx_vecs.append(plsc.load_gather(mx_ref, [ch]))

    trd = tr // 4
    rd = (tr * 2) % CB

    mbufs = [m0, m1]
    dbufs = [d0, d1]
    in_sems = [in_s0, in_s1]
    out_sems = [out_s0, out_s1]
    in_cps = [None, None]
    out_cps = [None, None]

    def start_load(i):
        b = i % 2
        t0 = tc0 + i * tc_c
        cpm = pltpu.async_copy(
            main_hbm.at[tr, pl.ds(t0, tc_c), :, :], mbufs[b], in_sems[b])
        cpd = pltpu.async_copy(
            dup_hbm.at[trd, pl.ds(t0, tc_c), pl.ds(rd, 2), :],
            dbufs[b], in_sems[b])
        in_cps[b] = (cpm, cpd)

    inf = jnp.float32(jnp.inf)
    zero = jnp.float32(0.0)

    def compute(mb, db):
        @plsc.parallel_loop(0, tc_c * (TB // L), unroll=1)
        def vec_body(i):
            t = i // (TB // L)
            j = i % (TB // L)
            sl = pl.ds(j * L, L)
            for r in range(CB):
                mnv = mn_vecs[r]
                mxv = mx_vecs[r]
                if r % 4 == 0:
                    m = mb[t, r, sl]
                    d = db[t, r // 4, sl]
                    mval = (m >= mnv) & (m <= mxv)
                    dval = (d >= mnv) & (d <= mxv)
                    v = jnp.minimum(jnp.where(mval, m, inf),
                                    jnp.where(dval, d, inf))
                    mb[t, r, sl] = jnp.where(mval | dval, v, zero)
                else:
                    x = mb[t, r, sl]
                    keep = (x >= mnv) & (x <= mxv)
                    mb[t, r, sl] = jnp.where(keep, x, zero)

    start_load(0)
    for i in range(nchunks):
        b = i % 2
        if i + 1 < nchunks:
            nb = (i + 1) % 2
            if i >= 1:
                out_cps[nb].wait()
            start_load(i + 1)
        in_cps[b][0].wait()
        in_cps[b][1].wait()
        compute(mbufs[b], dbufs[b])
        out_cps[b] = pltpu.async_copy(
            mbufs[b], out_hbm.at[tr, pl.ds(tc0 + i * tc_c, tc_c), :, :],
            out_sems[b])
    out_cps[(nchunks - 1) % 2].wait()
    if nchunks >= 2:
        out_cps[nchunks % 2].wait()


def kernel(main_out, dup_out, min_vals, max_vals):
    B = main_out.shape[0]
    nt = B // TB
    tc_w = nt // NQ
    tc_c = min(tc_w, 32)
    nchunks = tc_w // tc_c

    main4 = main_out.reshape(nt, TB, CB, CB).transpose(2, 0, 3, 1)
    dup4 = dup_out.reshape(nt, TB, K // CB, CB).transpose(2, 0, 3, 1)

    mesh = plsc.VectorSubcoreMesh(core_axis_name="c", subcore_axis_name="s")
    body = functools.partial(_edac_body, tc_w, tc_c, nchunks)
    f = pl.kernel(
        body,
        out_type=jax.ShapeDtypeStruct((CB, nt, CB, TB), jnp.float32),
        mesh=mesh,
        compiler_params=pltpu.CompilerParams(needs_layout_passes=False),
        scratch_types=[
            pltpu.VMEM((tc_c, CB, TB), jnp.float32),
            pltpu.VMEM((tc_c, CB, TB), jnp.float32),
            pltpu.VMEM((tc_c, 2, TB), jnp.float32),
            pltpu.VMEM((tc_c, 2, TB), jnp.float32),
            pltpu.VMEM((C,), jnp.float32),
            pltpu.VMEM((C,), jnp.float32),
            pltpu.SemaphoreType.DMA,
            pltpu.SemaphoreType.DMA,
            pltpu.SemaphoreType.DMA,
            pltpu.SemaphoreType.DMA,
        ],
    )
    out4 = f(main4, dup4, min_vals, max_vals)
    return out4.transpose(1, 3, 0, 2).reshape(B, C)
